# baseline (speedup 1.0000x reference)
import functools

import jax
import jax.numpy as jnp
from jax import lax
from jax.experimental import pallas as pl
from jax.experimental.pallas import tpu as pltpu
from jax.experimental.pallas import tpu_sc as plsc

B = 16384
D = 64
CHUNK = 128


def _pair_gather(uix_a, rix_a, gix_a, ut2, rt2, it2):
    info = plsc.get_sparse_core_info()
    nc, ns = info.num_cores, info.num_subcores
    nw = nc * ns
    bpw = B // nw
    nch = bpw // CHUNK
    mesh = plsc.VectorSubcoreMesh(core_axis_name="c", subcore_axis_name="s")

    @functools.partial(
        pl.kernel,
        mesh=mesh,
        out_type=(
            jax.ShapeDtypeStruct((B, 2 * D), jnp.float32),
            jax.ShapeDtypeStruct((B, 2 * D), jnp.float32),
            jax.ShapeDtypeStruct((B, 2 * D), jnp.float32),
        ),
        scratch_types=[
            pltpu.VMEM((bpw,), jnp.int32),
            pltpu.VMEM((bpw,), jnp.int32),
            pltpu.VMEM((bpw,), jnp.int32),
            pltpu.VMEM((CHUNK, 2 * D), jnp.float32),
            pltpu.VMEM((CHUNK, 2 * D), jnp.float32),
            pltpu.VMEM((CHUNK, 2 * D), jnp.float32),
            pltpu.SemaphoreType.DMA,
        ],
    )
    def k(uid_h, rid_h, ing_h, ut_h, rt_h, it_h, out_u, out_r, out_g,
          uix, rix, gix, ub, rb, gb, sem):
        wid = lax.axis_index("s") * nc + lax.axis_index("c")
        base = wid * bpw
        pltpu.sync_copy(uid_h.at[pl.ds(base, bpw)], uix)
        pltpu.sync_copy(rid_h.at[pl.ds(base, bpw)], rix)
        pltpu.sync_copy(ing_h.at[pl.ds(base, bpw)], gix)
        for j in range(nch):
            off = j * CHUNK
            sl = pl.ds(off, CHUNK)
            cu = pltpu.async_copy(ut_h.at[uix.at[sl]], ub, sem)
            cr = pltpu.async_copy(rt_h.at[rix.at[sl]], rb, sem)
            cg = pltpu.async_copy(it_h.at[gix.at[sl]], gb, sem)
            osl = pl.ds(base + off, CHUNK)
            cu.wait()
            pltpu.sync_copy(ub, out_u.at[osl])
            cr.wait()
            pltpu.sync_copy(rb, out_r.at[osl])
            cg.wait()
            pltpu.sync_copy(gb, out_g.at[osl])

    return k(uix_a, rix_a, gix_a, ut2, rt2, it2)


def kernel(uid, rid, ing, user_table, recipe_table, ingredient_table):
    ut2 = user_table.reshape(-1, 2 * D)
    rt2 = recipe_table.reshape(-1, 2 * D)
    it2 = ingredient_table.reshape(-1, 2 * D)
    pu, pr, pg = _pair_gather(
        uid >> 1, rid >> 1, ing >> 1, ut2, rt2, it2)
    su = jnp.where((uid & 1)[:, None] == 1, pu[:, D:], pu[:, :D])
    sr = jnp.where((rid & 1)[:, None] == 1, pr[:, D:], pr[:, :D])
    sg = jnp.where((ing & 1)[:, None] == 1, pg[:, D:], pg[:, :D])
    return jnp.concatenate((su, sr, sg), axis=1)


# speedup vs baseline: 1.9387x; 1.9387x over previous
import functools

import jax
import jax.numpy as jnp
from jax import lax
from jax.experimental import pallas as pl
from jax.experimental.pallas import tpu as pltpu
from jax.experimental.pallas import tpu_sc as plsc

B = 16384
D = 64
CHUNK = 32


def _gather_concat(uid, rid, ing, ut3, rt3, it3):
    info = plsc.get_sparse_core_info()
    nc, ns = info.num_cores, info.num_subcores
    nw = nc * ns
    bpw = B // nw
    nch = bpw // CHUNK
    mesh = plsc.VectorSubcoreMesh(core_axis_name="c", subcore_axis_name="s")

    @functools.partial(
        pl.kernel,
        mesh=mesh,
        out_type=jax.ShapeDtypeStruct((B, 3 * D), jnp.float32),
        scratch_types=[
            pltpu.SMEM((bpw,), jnp.int32),
            pltpu.SMEM((bpw,), jnp.int32),
            pltpu.SMEM((bpw,), jnp.int32),
            pltpu.VMEM_SHARED((B,), jnp.int32),
            pltpu.VMEM((CHUNK, 8, D), jnp.float32),
            pltpu.VMEM((CHUNK, 8, D), jnp.float32),
            pltpu.VMEM((CHUNK, 3 * D), jnp.float32),
            pltpu.SemaphoreType.DMA,
            pltpu.SemaphoreType.DMA,
        ],
    )
    def k(uid_h, rid_h, ing_h, ut_h, rt_h, it_h, out_h,
          smem_u, smem_r, smem_g, sp, tb0, tb1, asm, sem0, sem1):
        wid = lax.axis_index("s") * nc + lax.axis_index("c")
        base = wid * bpw
        bsl = pl.ds(base, bpw)

        for ix_h, smem in ((uid_h, smem_u), (rid_h, smem_r), (ing_h, smem_g)):
            pltpu.sync_copy(ix_h.at[bsl], sp.at[bsl])
            pltpu.sync_copy(sp.at[bsl], smem)

        tabs = (ut_h, rt_h, it_h)
        smems = (smem_u, smem_r, smem_g)
        tbufs = (tb0, tb1)
        sems = (sem0, sem1)

        def fire(s):
            k_, t = divmod(s, 3)
            tab, smem = tabs[t], smems[t]
            buf, sem = tbufs[s % 2], sems[s % 2]
            off = k_ * CHUNK

            def body(i, carry):
                blk = smem[off + i] >> 3
                pltpu.async_copy(tab.at[blk], buf.at[i], sem)
                return carry

            lax.fori_loop(0, CHUNK, body, 0)

        def drain_and_select(s):
            k_, t = divmod(s, 3)
            tab, smem = tabs[t], smems[t]
            buf, sem = tbufs[s % 2], sems[s % 2]
            pltpu.make_async_copy(tab.at[pl.ds(0, CHUNK)], buf, sem).wait()
            off = k_ * CHUNK

            def body(i, carry):
                j = smem[off + i] & 7
                for cg in range(D // 16):
                    csl = pl.ds(cg * 16, 16)
                    asm[i, pl.ds(t * D + cg * 16, 16)] = buf[i, j, csl]
                return carry

            lax.fori_loop(0, CHUNK, body, 0)

        nsteps = nch * 3
        fire(0)
        for s in range(nsteps):
            if s + 1 < nsteps:
                fire(s + 1)
            drain_and_select(s)
            k_, t = divmod(s, 3)
            if t == 2:
                pltpu.sync_copy(
                    asm, out_h.at[pl.ds(base + k_ * CHUNK, CHUNK)])

    return k(uid, rid, ing, ut3, rt3, it3)


def kernel(uid, rid, ing, user_table, recipe_table, ingredient_table):
    ut3 = user_table.reshape(-1, 8, D)
    rt3 = recipe_table.reshape(-1, 8, D)
    it3 = ingredient_table.reshape(-1, 8, D)
    return _gather_concat(uid, rid, ing, ut3, rt3, it3)
